# Initial kernel scaffold; baseline (speedup 1.0000x reference)
import functools

import jax
import jax.numpy as jnp
from jax import lax
from jax.experimental import pallas as pl
from jax.experimental.pallas import tpu as pltpu
from jax.experimental.pallas import tpu_sc as plsc

N_NODES = 10000
N_EDGES = 320000
D = 128

NC = 2
NS = 16
NW = NC * NS
CHUNK = 128
N_CHUNKS = N_EDGES // CHUNK
FULL_ROUNDS = N_CHUNKS // NW
TAIL = N_CHUNKS - FULL_ROUNDS * NW
ROWS_PER_TILE = N_NODES // NS
RP5 = ROWS_PER_TILE // 5


def _node_pre_body(x_ref, w1_ref, w2_ref, xws_ref, xwd_ref):
    xb = x_ref[...]
    xws_ref[...] = jnp.dot(xb, w1_ref[...], preferred_element_type=jnp.float32)
    xwd_ref[...] = jnp.dot(xb, w2_ref[...], preferred_element_type=jnp.float32)


def _ea_body(ea_in_ref, w3_ref, be_ref, ea_ref):
    ea_ref[...] = (
        jnp.dot(ea_in_ref[...], w3_ref[...], preferred_element_type=jnp.float32)
        + be_ref[...]
    )


def _node_mlp_body(x_ref, p0_ref, p1_ref, w1_ref, w2_ref, bn_ref, out_ref):
    acc = jnp.dot(x_ref[...], w1_ref[...], preferred_element_type=jnp.float32)
    agg = p0_ref[...] + p1_ref[...]
    acc = acc + jnp.dot(agg, w2_ref[...], preferred_element_type=jnp.float32)
    out_ref[...] = jnp.maximum(acc + bn_ref[...], 0.0)


def _sc_edge_body(xws_hbm, xwd_hbm, ea_hbm, src_hbm, dst_hbm,
                  ue_hbm, part_hbm,
                  src_v, dst_v, rows_s, rows_d, ea_v,
                  acc_sh, sem1, sem2, sem3):
    cid = lax.axis_index("c")
    sid = lax.axis_index("s")
    wid = cid * NS + sid

    def zrow(r, _):
        for c in range(D // 16):
            ea_v[r, pl.ds(c * 16, 16)] = jnp.zeros((16,), jnp.float32)
        return 0
    lax.fori_loop(0, CHUNK, zrow, 0)

    for m in range(ROWS_PER_TILE // RP5):
        o2 = sid * ROWS_PER_TILE + m * RP5
        pltpu.sync_copy(ea_v.at[pl.ds(0, RP5), :], acc_sh.at[pl.ds(o2, RP5), :])
    plsc.subcore_barrier()

    def process_chunk(chunk_id):
        off = pl.multiple_of(chunk_id * CHUNK, CHUNK)
        pltpu.sync_copy(src_hbm.at[pl.ds(off, CHUNK)], src_v)
        pltpu.sync_copy(dst_hbm.at[pl.ds(off, CHUNK)], dst_v)
        cp1 = pltpu.async_copy(xws_hbm.at[src_v], rows_s, sem1)
        cp2 = pltpu.async_copy(xwd_hbm.at[dst_v], rows_d, sem2)
        cp3 = pltpu.async_copy(ea_hbm.at[pl.ds(off, CHUNK), :], ea_v, sem3)
        cp1.wait()
        cp2.wait()
        cp3.wait()

        def crow(r, _):
            for c in range(D // 16):
                s = pl.ds(c * 16, 16)
                v = rows_s[r, s] + rows_d[r, s] + ea_v[r, s]
                ea_v[r, s] = jnp.maximum(v, 0.0)
            return 0
        lax.fori_loop(0, CHUNK, crow, 0)

        pltpu.sync_copy(ea_v, ue_hbm.at[pl.ds(off, CHUNK), :])
        pltpu.sync_copy(ea_v, acc_sh.at[dst_v], add=True)

    def round_body(t, _):
        process_chunk(t * NW + wid)
        return 0
    lax.fori_loop(0, FULL_ROUNDS, round_body, 0)

    @pl.when(wid < TAIL)
    def _():
        process_chunk(FULL_ROUNDS * NW + wid)

    plsc.subcore_barrier()
    for m in range(ROWS_PER_TILE // RP5):
        o2 = sid * ROWS_PER_TILE + m * RP5
        pltpu.sync_copy(acc_sh.at[pl.ds(o2, RP5), :], rows_s.at[pl.ds(0, RP5), :])
        pltpu.sync_copy(rows_s.at[pl.ds(0, RP5), :],
                        part_hbm.at[pl.ds(cid * N_NODES + o2, RP5), :])


_sc_edge_kernel = functools.partial(
    pl.kernel,
    out_type=[
        jax.ShapeDtypeStruct((N_EDGES, D), jnp.float32),
        jax.ShapeDtypeStruct((NC * N_NODES, D), jnp.float32),
    ],
    mesh=plsc.VectorSubcoreMesh(core_axis_name="c", subcore_axis_name="s"),
    scratch_types=[
        pltpu.VMEM((CHUNK,), jnp.int32),
        pltpu.VMEM((CHUNK,), jnp.int32),
        pltpu.VMEM((CHUNK, D), jnp.float32),
        pltpu.VMEM((CHUNK, D), jnp.float32),
        pltpu.VMEM((CHUNK, D), jnp.float32),
        pltpu.VMEM_SHARED((N_NODES, D), jnp.float32),
        pltpu.SemaphoreType.DMA,
        pltpu.SemaphoreType.DMA,
        pltpu.SemaphoreType.DMA,
    ],
)(_sc_edge_body)


def kernel(x, edge_index, edge_attr, W_e, b_e, W_n, b_n):
    x = x.astype(jnp.float32)
    src = edge_index[0].astype(jnp.int32)
    dst = edge_index[1].astype(jnp.int32)

    we1 = W_e[0:D]
    we2 = W_e[D:2 * D]
    we3 = W_e[2 * D:]
    be2 = b_e.reshape(1, D)
    wn1 = W_n[0:D]
    wn2 = W_n[D:]
    bn2 = b_n.reshape(1, D)

    nb = 2000
    xws, xwd = pl.pallas_call(
        _node_pre_body,
        grid=(N_NODES // nb,),
        in_specs=[
            pl.BlockSpec((nb, D), lambda i: (i, 0)),
            pl.BlockSpec((D, D), lambda i: (0, 0)),
            pl.BlockSpec((D, D), lambda i: (0, 0)),
        ],
        out_specs=[
            pl.BlockSpec((nb, D), lambda i: (i, 0)),
            pl.BlockSpec((nb, D), lambda i: (i, 0)),
        ],
        out_shape=[
            jax.ShapeDtypeStruct((N_NODES, D), jnp.float32),
            jax.ShapeDtypeStruct((N_NODES, D), jnp.float32),
        ],
    )(x, we1, we2)

    eb = 4000
    ea = pl.pallas_call(
        _ea_body,
        grid=(N_EDGES // eb,),
        in_specs=[
            pl.BlockSpec((eb, 16), lambda i: (i, 0)),
            pl.BlockSpec((16, D), lambda i: (0, 0)),
            pl.BlockSpec((1, D), lambda i: (0, 0)),
        ],
        out_specs=pl.BlockSpec((eb, D), lambda i: (i, 0)),
        out_shape=jax.ShapeDtypeStruct((N_EDGES, D), jnp.float32),
    )(edge_attr.astype(jnp.float32), we3, be2)

    updated_edge, partials = _sc_edge_kernel(xws, xwd, ea, src, dst)

    updated_node = pl.pallas_call(
        _node_mlp_body,
        grid=(N_NODES // nb,),
        in_specs=[
            pl.BlockSpec((nb, D), lambda i: (i, 0)),
            pl.BlockSpec((nb, D), lambda i: (i, 0)),
            pl.BlockSpec((nb, D), lambda i: (i + N_NODES // nb, 0)),
            pl.BlockSpec((D, D), lambda i: (0, 0)),
            pl.BlockSpec((D, D), lambda i: (0, 0)),
            pl.BlockSpec((1, D), lambda i: (0, 0)),
        ],
        out_specs=pl.BlockSpec((nb, D), lambda i: (i, 0)),
        out_shape=jax.ShapeDtypeStruct((N_NODES, D), jnp.float32),
    )(x, partials, partials, wn1, wn2, bn2)

    return (updated_node, updated_edge)


# speedup vs baseline: 3.2995x; 3.2995x over previous
import functools

import jax
import jax.numpy as jnp
from jax import lax
from jax.experimental import pallas as pl
from jax.experimental.pallas import tpu as pltpu
from jax.experimental.pallas import tpu_sc as plsc

N_NODES = 10000
N_EDGES = 320000
D = 128

NC = 2
NS = 16
NW = NC * NS
CHUNK = 128
N_CHUNKS = N_EDGES // CHUNK
FULL_ROUNDS = N_CHUNKS // NW
TAIL = N_CHUNKS - FULL_ROUNDS * NW
ACC_ROWS = 10112
ROWS_PER_TILE = ACC_ROWS // NS
GRANS = (128, 128, 128, 128, 120)


def _node_pre_body(x_ref, w1_ref, w2_ref, xws_ref, xwd_ref):
    xb = x_ref[...]
    xws_ref[...] = jnp.dot(xb, w1_ref[...], preferred_element_type=jnp.float32)
    xwd_ref[...] = jnp.dot(xb, w2_ref[...], preferred_element_type=jnp.float32)


def _ea_body(ea_in_ref, w3_ref, be_ref, ea_ref):
    ea_ref[...] = (
        jnp.dot(ea_in_ref[...], w3_ref[...], preferred_element_type=jnp.float32)
        + be_ref[...]
    )


def _node_mlp_body(x_ref, p0_ref, p1_ref, w1_ref, w2_ref, bn_ref, out_ref):
    acc = jnp.dot(x_ref[...], w1_ref[...], preferred_element_type=jnp.float32)
    agg = p0_ref[0] + p1_ref[0]
    acc = acc + jnp.dot(agg, w2_ref[...], preferred_element_type=jnp.float32)
    out_ref[...] = jnp.maximum(acc + bn_ref[...], 0.0)


def _sc_edge_body(xws_hbm, xwd_hbm, ea_hbm, src_hbm, dst_hbm,
                  ue_hbm, part_hbm,
                  src_v, dst_v, rows_s, rows_d, ea_v,
                  acc_sh, sem1, sem2, sem3):
    cid = lax.axis_index("c")
    sid = lax.axis_index("s")
    wid = cid * NS + sid

    def zrow(r, _):
        for c in range(D // 16):
            ea_v[r, pl.ds(c * 16, 16)] = jnp.zeros((16,), jnp.float32)
        return 0
    lax.fori_loop(0, CHUNK, zrow, 0)

    o2 = sid * ROWS_PER_TILE
    for g in GRANS:
        pltpu.sync_copy(ea_v.at[pl.ds(0, g), :], acc_sh.at[pl.ds(o2, g), :])
        o2 += g
    plsc.subcore_barrier()

    def process_chunk(chunk_id):
        off = pl.multiple_of(chunk_id * CHUNK, CHUNK)
        pltpu.sync_copy(src_hbm.at[pl.ds(off, CHUNK)], src_v)
        pltpu.sync_copy(dst_hbm.at[pl.ds(off, CHUNK)], dst_v)
        cp1 = pltpu.async_copy(xws_hbm.at[src_v], rows_s, sem1)
        cp2 = pltpu.async_copy(xwd_hbm.at[dst_v], rows_d, sem2)
        cp3 = pltpu.async_copy(ea_hbm.at[pl.ds(off, CHUNK), :], ea_v, sem3)
        cp1.wait()
        cp2.wait()
        cp3.wait()

        def crow(r, _):
            for c in range(D // 16):
                s = pl.ds(c * 16, 16)
                v = rows_s[r, s] + rows_d[r, s] + ea_v[r, s]
                ea_v[r, s] = jnp.maximum(v, 0.0)
            return 0
        lax.fori_loop(0, CHUNK, crow, 0)

        pltpu.sync_copy(ea_v, ue_hbm.at[pl.ds(off, CHUNK), :])
        pltpu.sync_copy(ea_v, acc_sh.at[dst_v], add=True)

    def round_body(t, _):
        process_chunk(t * NW + wid)
        return 0
    lax.fori_loop(0, FULL_ROUNDS, round_body, 0)

    @pl.when(wid < TAIL)
    def _():
        process_chunk(FULL_ROUNDS * NW + wid)

    plsc.subcore_barrier()
    o2 = sid * ROWS_PER_TILE
    for g in GRANS:
        pltpu.sync_copy(acc_sh.at[pl.ds(o2, g), :], rows_s.at[pl.ds(0, g), :])
        pltpu.sync_copy(rows_s.at[pl.ds(0, g), :],
                        part_hbm.at[cid, pl.ds(o2, g), :])
        o2 += g


_sc_edge_kernel = functools.partial(
    pl.kernel,
    out_type=[
        jax.ShapeDtypeStruct((N_EDGES, D), jnp.float32),
        jax.ShapeDtypeStruct((NC, ACC_ROWS, D), jnp.float32),
    ],
    mesh=plsc.VectorSubcoreMesh(core_axis_name="c", subcore_axis_name="s"),
    scratch_types=[
        pltpu.VMEM((CHUNK,), jnp.int32),
        pltpu.VMEM((CHUNK,), jnp.int32),
        pltpu.VMEM((CHUNK, D), jnp.float32),
        pltpu.VMEM((CHUNK, D), jnp.float32),
        pltpu.VMEM((CHUNK, D), jnp.float32),
        pltpu.VMEM_SHARED((ACC_ROWS, D), jnp.float32),
        pltpu.SemaphoreType.DMA,
        pltpu.SemaphoreType.DMA,
        pltpu.SemaphoreType.DMA,
    ],
)(_sc_edge_body)


def kernel(x, edge_index, edge_attr, W_e, b_e, W_n, b_n):
    x = x.astype(jnp.float32)
    src = edge_index[0].astype(jnp.int32)
    dst = edge_index[1].astype(jnp.int32)

    we1 = W_e[0:D]
    we2 = W_e[D:2 * D]
    we3 = W_e[2 * D:]
    be2 = b_e.reshape(1, D)
    wn1 = W_n[0:D]
    wn2 = W_n[D:]
    bn2 = b_n.reshape(1, D)

    nb = 2000
    xws, xwd = pl.pallas_call(
        _node_pre_body,
        grid=(N_NODES // nb,),
        in_specs=[
            pl.BlockSpec((nb, D), lambda i: (i, 0)),
            pl.BlockSpec((D, D), lambda i: (0, 0)),
            pl.BlockSpec((D, D), lambda i: (0, 0)),
        ],
        out_specs=[
            pl.BlockSpec((nb, D), lambda i: (i, 0)),
            pl.BlockSpec((nb, D), lambda i: (i, 0)),
        ],
        out_shape=[
            jax.ShapeDtypeStruct((N_NODES, D), jnp.float32),
            jax.ShapeDtypeStruct((N_NODES, D), jnp.float32),
        ],
    )(x, we1, we2)

    eb = 4000
    ea = pl.pallas_call(
        _ea_body,
        grid=(N_EDGES // eb,),
        in_specs=[
            pl.BlockSpec((eb, 16), lambda i: (i, 0)),
            pl.BlockSpec((16, D), lambda i: (0, 0)),
            pl.BlockSpec((1, D), lambda i: (0, 0)),
        ],
        out_specs=pl.BlockSpec((eb, D), lambda i: (i, 0)),
        out_shape=jax.ShapeDtypeStruct((N_EDGES, D), jnp.float32),
    )(edge_attr.astype(jnp.float32), we3, be2)

    updated_edge, partials = _sc_edge_kernel(xws, xwd, ea, src, dst)

    updated_node = pl.pallas_call(
        _node_mlp_body,
        grid=(N_NODES // nb,),
        in_specs=[
            pl.BlockSpec((nb, D), lambda i: (i, 0)),
            pl.BlockSpec((1, nb, D), lambda i: (0, i, 0)),
            pl.BlockSpec((1, nb, D), lambda i: (1, i, 0)),
            pl.BlockSpec((D, D), lambda i: (0, 0)),
            pl.BlockSpec((D, D), lambda i: (0, 0)),
            pl.BlockSpec((1, D), lambda i: (0, 0)),
        ],
        out_specs=pl.BlockSpec((nb, D), lambda i: (i, 0)),
        out_shape=jax.ShapeDtypeStruct((N_NODES, D), jnp.float32),
    )(x, partials, partials, wn1, wn2, bn2)

    return (updated_node, updated_edge)


# speedup vs baseline: 4.9088x; 1.4877x over previous
import functools

import jax
import jax.numpy as jnp
from jax import lax
from jax.experimental import pallas as pl
from jax.experimental.pallas import tpu as pltpu
from jax.experimental.pallas import tpu_sc as plsc

N_NODES = 10000
N_EDGES = 320000
D = 128

NC = 2
NS = 16
NW = NC * NS
CHUNK = 128
N_CHUNKS = N_EDGES // CHUNK
FULL_ROUNDS = N_CHUNKS // NW
TAIL = N_CHUNKS - FULL_ROUNDS * NW
ACC_ROWS = 10112
ROWS_PER_TILE = ACC_ROWS // NS
GRANS = (128, 128, 128, 128, 120)


def _node_pre_body(x_ref, w1_ref, w2_ref, xws_ref, xwd_ref):
    xb = x_ref[...]
    xws_ref[...] = jnp.dot(xb, w1_ref[...], preferred_element_type=jnp.float32)
    xwd_ref[...] = jnp.dot(xb, w2_ref[...], preferred_element_type=jnp.float32)


def _ea_body(ea_in_ref, w3_ref, be_ref, ea_ref):
    ea_ref[...] = (
        jnp.dot(ea_in_ref[...], w3_ref[...], preferred_element_type=jnp.float32)
        + be_ref[...]
    )


def _node_mlp_body(x_ref, p0_ref, p1_ref, w1_ref, w2_ref, bn_ref, out_ref):
    acc = jnp.dot(x_ref[...], w1_ref[...], preferred_element_type=jnp.float32)
    agg = p0_ref[0] + p1_ref[0]
    acc = acc + jnp.dot(agg, w2_ref[...], preferred_element_type=jnp.float32)
    out_ref[...] = jnp.maximum(acc + bn_ref[...], 0.0)


def _sc_edge_body(xws_hbm, xwd_hbm, ea_hbm, idx_hbm,
                  ue_hbm, part_hbm,
                  idx0, idx1,
                  buf0, buf1,
                  acc_sh,
                  se0, si0, sg0, sd0, su0, sa0,
                  se1, si1, sg1, sd1, su1, sa1):
    cid = lax.axis_index("c")
    sid = lax.axis_index("s")
    wid = cid * NS + sid

    def zrow(r, _):
        for c in range(D // 16):
            buf0[r, pl.ds(c * 16, 16)] = jnp.zeros((16,), jnp.float32)
        return 0
    lax.fori_loop(0, CHUNK, zrow, 0)

    o2 = sid * ROWS_PER_TILE
    for g in GRANS:
        pltpu.sync_copy(buf0.at[pl.ds(0, g), :], acc_sh.at[pl.ds(o2, g), :])
        o2 += g
    plsc.subcore_barrier()

    bufs = ((buf0, idx0, se0, si0, sg0, sd0, su0, sa0),
            (buf1, idx1, se1, si1, sg1, sd1, su1, sa1))

    def off_of(t):
        return pl.multiple_of((t * NW + wid) * CHUNK, CHUNK)

    def issue_e(t, b):
        ev, ix, se, si = bufs[b][0], bufs[b][1], bufs[b][2], bufs[b][3]
        pltpu.async_copy(ea_hbm.at[pl.ds(off_of(t), CHUNK), :], ev, se)
        pltpu.async_copy(idx_hbm.at[wid * SLOTS + t], ix, si)

    def wait_e(t, b):
        ev, ix, se, si = bufs[b][0], bufs[b][1], bufs[b][2], bufs[b][3]
        pltpu.make_async_copy(ea_hbm.at[pl.ds(off_of(t), CHUNK), :], ev, se).wait()
        pltpu.make_async_copy(idx_hbm.at[wid * SLOTS + t], ix, si).wait()

    def issue_g(t, b):
        ev, ix, sg, sdm = bufs[b][0], bufs[b][1], bufs[b][4], bufs[b][5]
        pltpu.async_copy(xws_hbm.at[ix.at[0]], ev, sg, add=True)
        pltpu.async_copy(xwd_hbm.at[ix.at[1]], ev, sdm, add=True)

    def wait_g(t, b):
        ev, ix, sg, sdm = bufs[b][0], bufs[b][1], bufs[b][4], bufs[b][5]
        pltpu.make_async_copy(xws_hbm.at[ix.at[0]], ev, sg).wait()
        pltpu.make_async_copy(xwd_hbm.at[ix.at[1]], ev, sdm).wait()

    def compute(b):
        ev = bufs[b][0]

        @plsc.parallel_loop(0, CHUNK)
        def _(r):
            for c in range(D // 16):
                s = pl.ds(c * 16, 16)
                ev[r, s] = jnp.maximum(ev[r, s], 0.0)

    def issue_w(t, b):
        ev, ix, su, sa = bufs[b][0], bufs[b][1], bufs[b][6], bufs[b][7]
        pltpu.async_copy(ev, ue_hbm.at[pl.ds(off_of(t), CHUNK), :], su)
        pltpu.async_copy(ev, acc_sh.at[ix.at[1]], sa, add=True)

    def wait_w(t, b):
        ev, ix, su, sa = bufs[b][0], bufs[b][1], bufs[b][6], bufs[b][7]
        pltpu.make_async_copy(ev, ue_hbm.at[pl.ds(off_of(t), CHUNK), :], su).wait()
        pltpu.make_async_copy(ev, acc_sh.at[ix.at[1]], sa).wait()

    def step(t, k):
        @pl.when(t >= 2)
        def _():
            wait_w(t - 2, k)
        issue_e(t, k)

        @pl.when(t >= 1)
        def _():
            wait_g(t - 1, 1 - k)
            compute(1 - k)
            issue_w(t - 1, 1 - k)
        wait_e(t, k)
        issue_g(t, k)

    def body(u, _):
        step(2 * u, 0)
        step(2 * u + 1, 1)
        return 0
    lax.fori_loop(0, FULL_ROUNDS // 2, body, 0)
    wait_g(FULL_ROUNDS - 1, 1)
    compute(1)
    issue_w(FULL_ROUNDS - 1, 1)
    wait_w(FULL_ROUNDS - 2, 0)
    wait_w(FULL_ROUNDS - 1, 1)

    @pl.when(wid < TAIL)
    def _():
        t = FULL_ROUNDS
        issue_e(t, 0)
        wait_e(t, 0)
        issue_g(t, 0)
        wait_g(t, 0)
        compute(0)
        issue_w(t, 0)
        wait_w(t, 0)

    plsc.subcore_barrier()
    o2 = sid * ROWS_PER_TILE
    for g in GRANS:
        pltpu.sync_copy(acc_sh.at[pl.ds(o2, g), :], buf0.at[pl.ds(0, g), :])
        pltpu.sync_copy(buf0.at[pl.ds(0, g), :],
                        part_hbm.at[cid, pl.ds(o2, g), :])
        o2 += g


SLOTS = 80

_sc_edge_kernel = functools.partial(
    pl.kernel,
    out_type=[
        jax.ShapeDtypeStruct((N_EDGES, D), jnp.float32),
        jax.ShapeDtypeStruct((NC, ACC_ROWS, D), jnp.float32),
    ],
    mesh=plsc.VectorSubcoreMesh(core_axis_name="c", subcore_axis_name="s"),
    scratch_types=[
        pltpu.VMEM((2, CHUNK), jnp.int32),
        pltpu.VMEM((2, CHUNK), jnp.int32),
        pltpu.VMEM((CHUNK, D), jnp.float32),
        pltpu.VMEM((CHUNK, D), jnp.float32),
        pltpu.VMEM_SHARED((ACC_ROWS, D), jnp.float32),
    ] + [pltpu.SemaphoreType.DMA] * 12,
)(_sc_edge_body)


def kernel(x, edge_index, edge_attr, W_e, b_e, W_n, b_n):
    x = x.astype(jnp.float32)
    src = edge_index[0].astype(jnp.int32)
    dst = edge_index[1].astype(jnp.int32)

    we1 = W_e[0:D]
    we2 = W_e[D:2 * D]
    we3 = W_e[2 * D:]
    be2 = b_e.reshape(1, D)
    wn1 = W_n[0:D]
    wn2 = W_n[D:]
    bn2 = b_n.reshape(1, D)

    nb = 2000
    xws, xwd = pl.pallas_call(
        _node_pre_body,
        grid=(N_NODES // nb,),
        in_specs=[
            pl.BlockSpec((nb, D), lambda i: (i, 0)),
            pl.BlockSpec((D, D), lambda i: (0, 0)),
            pl.BlockSpec((D, D), lambda i: (0, 0)),
        ],
        out_specs=[
            pl.BlockSpec((nb, D), lambda i: (i, 0)),
            pl.BlockSpec((nb, D), lambda i: (i, 0)),
        ],
        out_shape=[
            jax.ShapeDtypeStruct((N_NODES, D), jnp.float32),
            jax.ShapeDtypeStruct((N_NODES, D), jnp.float32),
        ],
    )(x, we1, we2)

    eb = 4000
    ea = pl.pallas_call(
        _ea_body,
        grid=(N_EDGES // eb,),
        in_specs=[
            pl.BlockSpec((eb, 16), lambda i: (i, 0)),
            pl.BlockSpec((16, D), lambda i: (0, 0)),
            pl.BlockSpec((1, D), lambda i: (0, 0)),
        ],
        out_specs=pl.BlockSpec((eb, D), lambda i: (i, 0)),
        out_shape=jax.ShapeDtypeStruct((N_EDGES, D), jnp.float32),
    )(edge_attr.astype(jnp.float32), we3, be2)

    pad = jnp.zeros((SLOTS * NW - N_CHUNKS, CHUNK), jnp.int32)
    src_w = jnp.concatenate([src.reshape(N_CHUNKS, CHUNK), pad]) \
        .reshape(SLOTS, NW, CHUNK).transpose(1, 0, 2)
    dst_w = jnp.concatenate([dst.reshape(N_CHUNKS, CHUNK), pad]) \
        .reshape(SLOTS, NW, CHUNK).transpose(1, 0, 2)
    idx_w = jnp.stack([src_w, dst_w], axis=2).reshape(NW * SLOTS, 2, CHUNK)

    updated_edge, partials = _sc_edge_kernel(xws, xwd, ea, idx_w)

    updated_node = pl.pallas_call(
        _node_mlp_body,
        grid=(N_NODES // nb,),
        in_specs=[
            pl.BlockSpec((nb, D), lambda i: (i, 0)),
            pl.BlockSpec((1, nb, D), lambda i: (0, i, 0)),
            pl.BlockSpec((1, nb, D), lambda i: (1, i, 0)),
            pl.BlockSpec((D, D), lambda i: (0, 0)),
            pl.BlockSpec((D, D), lambda i: (0, 0)),
            pl.BlockSpec((1, D), lambda i: (0, 0)),
        ],
        out_specs=pl.BlockSpec((nb, D), lambda i: (i, 0)),
        out_shape=jax.ShapeDtypeStruct((N_NODES, D), jnp.float32),
    )(x, partials, partials, wn1, wn2, bn2)

    return (updated_node, updated_edge)
